# baseline (speedup 1.0000x reference)
import functools

import jax
import jax.numpy as jnp
from jax import lax
from jax.experimental import pallas as pl
from jax.experimental.pallas import tpu as pltpu
from jax.experimental.pallas import tpu_sc as plsc

BATCH = 16384
NF = 32

_info = plsc.get_sparse_core_info()
_NC, _NS, _L = _info.num_cores, _info.num_subcores, _info.num_lanes
NW = _NC * _NS
B_PER_W = BATCH // NW
NCHUNK = 4
CH = B_PER_W // NCHUNK

_mesh = plsc.VectorSubcoreMesh(core_axis_name="c", subcore_axis_name="s")


@functools.partial(
    pl.kernel,
    mesh=_mesh,
    out_type=jax.ShapeDtypeStruct((BATCH,), jnp.float32),
    compiler_params=pltpu.CompilerParams(needs_layout_passes=False,
                                         use_tc_tiling_on_sc=False),
    scratch_types=[
        pltpu.VMEM((NCHUNK, CH), jnp.int32),
        pltpu.VMEM((NCHUNK, CH), jnp.int32),
        pltpu.VMEM((NF, NCHUNK, CH), jnp.float32),
        pltpu.VMEM((NF, NCHUNK, CH), jnp.float32),
        pltpu.VMEM((NCHUNK, CH), jnp.float32),
        pltpu.SemaphoreType.DMA,
    ],
)
def _mf_kernel(user_hbm, business_hbm, uft_hbm, bft_hbm, out_hbm,
               uidx, bidx, ubuf, bbuf, outv, sem):
    wid = lax.axis_index("s") * _NC + lax.axis_index("c")
    base = wid * B_PER_W

    for j in range(NCHUNK):
        pltpu.sync_copy(user_hbm.at[pl.ds(base + j * CH, CH)], uidx.at[j])
        pltpu.sync_copy(business_hbm.at[pl.ds(base + j * CH, CH)], bidx.at[j])

    copies = []
    for j in range(NCHUNK):
        for c in range(NF):
            copies.append(pltpu.async_copy(
                uft_hbm.at[c].at[uidx.at[j]], ubuf.at[c, j], sem))
            copies.append(pltpu.async_copy(
                bft_hbm.at[c].at[bidx.at[j]], bbuf.at[c, j], sem))
    for cp in copies:
        cp.wait()

    for j in range(NCHUNK):
        def group_body(g, _, j=j):
            def c_body(c, acc, j=j):
                uv = ubuf[c, j, pl.ds(g * _L, _L)]
                bv = bbuf[c, j, pl.ds(g * _L, _L)]
                return acc + uv * bv

            acc = lax.fori_loop(0, NF, c_body, jnp.zeros((_L,), jnp.float32))
            outv[j, pl.ds(g * _L, _L)] = acc
            return 0

        lax.fori_loop(0, CH // _L, group_body, 0)

    for j in range(NCHUNK):
        pltpu.sync_copy(outv.at[j], out_hbm.at[pl.ds(base + j * CH, CH)])


def kernel(user, business, user_factors, business_factors):
    return _mf_kernel(user, business, user_factors.T, business_factors.T)


# speedup vs baseline: 20.9998x; 20.9998x over previous
import functools

import jax
import jax.numpy as jnp
from jax import lax
from jax.experimental import pallas as pl
from jax.experimental.pallas import tpu as pltpu
from jax.experimental.pallas import tpu_sc as plsc

BATCH = 16384
NF = 32
TILE = 128
PITCH = 137

_info = plsc.get_sparse_core_info()
_NC, _NS, _L = _info.num_cores, _info.num_subcores, _info.num_lanes
NW = _NC * _NS
B_PER_W = BATCH // NW
RING = 4

_mesh = plsc.VectorSubcoreMesh(core_axis_name="c", subcore_axis_name="s")

_scratch = [
    pltpu.VMEM((B_PER_W,), jnp.int32),
    pltpu.VMEM((B_PER_W,), jnp.int32),
    pltpu.SMEM((B_PER_W,), jnp.int32),
    pltpu.SMEM((B_PER_W,), jnp.int32),
    pltpu.VMEM((B_PER_W,), jnp.float32),
]
for _ in range(RING):
    _scratch.append(pltpu.VMEM((NF, PITCH), jnp.float32))
    _scratch.append(pltpu.VMEM((NF, PITCH), jnp.float32))
    _scratch.append(pltpu.SemaphoreType.DMA)


@functools.partial(
    pl.kernel,
    mesh=_mesh,
    out_type=jax.ShapeDtypeStruct((BATCH,), jnp.float32),
    compiler_params=pltpu.CompilerParams(needs_layout_passes=False),
    scratch_types=_scratch,
)
def _mf_kernel(user_hbm, business_hbm, uft_hbm, bft_hbm, out_hbm,
               uidx_v, bidx_v, uidx, bidx, outv, *ring):
    wid = lax.axis_index("s") * _NC + lax.axis_index("c")
    base = wid * B_PER_W

    pltpu.sync_copy(user_hbm.at[pl.ds(base, B_PER_W)], uidx_v)
    pltpu.sync_copy(business_hbm.at[pl.ds(base, B_PER_W)], bidx_v)

    for g in range(B_PER_W // _L):
        vu = uidx_v[pl.ds(g * _L, _L)]
        vb = bidx_v[pl.ds(g * _L, _L)]
        for m in range(_L):
            uidx[g * _L + m] = vu[m]
            bidx[g * _L + m] = vb[m]

    ubufs = ring[0::3]
    bbufs = ring[1::3]
    sems = ring[2::3]
    lanes = lax.iota(jnp.int32, _L)

    def start(i, b):
        ru = uidx[i]
        rb = bidx[i]
        tu = pl.multiple_of(lax.bitwise_and(ru, -TILE), TILE)
        tb = pl.multiple_of(lax.bitwise_and(rb, -TILE), TILE)
        pltpu.async_copy(uft_hbm.at[pl.ds(0, NF), pl.ds(tu, TILE)],
                         ubufs[b].at[pl.ds(0, NF), pl.ds(0, TILE)], sems[b])
        pltpu.async_copy(bft_hbm.at[pl.ds(0, NF), pl.ds(tb, TILE)],
                         bbufs[b].at[pl.ds(0, NF), pl.ds(0, TILE)], sems[b])

    def drain(b):
        dummy = uft_hbm.at[pl.ds(0, NF), pl.ds(0, TILE)]
        pltpu.make_async_copy(
            dummy, ubufs[b].at[pl.ds(0, NF), pl.ds(0, TILE)], sems[b]).wait()
        pltpu.make_async_copy(
            dummy, bbufs[b].at[pl.ds(0, NF), pl.ds(0, TILE)], sems[b]).wait()

    def compute(i, b, acc):
        lu = lax.bitwise_and(uidx[i], TILE - 1)
        lb = lax.bitwise_and(bidx[i], TILE - 1)
        lu_v = jnp.full((_L,), lu, jnp.int32)
        lb_v = jnp.full((_L,), lb, jnp.int32)
        u_lo = plsc.load_gather(ubufs[b], [lanes, lu_v])
        u_hi = plsc.load_gather(ubufs[b], [lanes + _L, lu_v])
        b_lo = plsc.load_gather(bbufs[b], [lanes, lb_v])
        b_hi = plsc.load_gather(bbufs[b], [lanes + _L, lb_v])
        prod = u_lo * b_lo + u_hi * b_hi
        s = jnp.full((_L,), jnp.sum(prod), jnp.float32)
        return jnp.where(lanes == lax.bitwise_and(i, _L - 1), s, acc)

    for b in range(RING):
        start(b, b)

    group = _L // RING

    def macro_body(k, acc):
        for b in range(RING):
            i = k * RING + b
            drain(b)
            acc = compute(i, b, acc)

            @pl.when(i + RING < B_PER_W)
            def _():
                start(i + RING, b)

        flush = lax.bitwise_and(k, group - 1) == group - 1

        @pl.when(flush)
        def _():
            outv[pl.ds((k // group) * _L, _L)] = acc

        return jnp.where(flush, jnp.zeros((_L,), jnp.float32), acc)

    lax.fori_loop(0, B_PER_W // RING, macro_body,
                  jnp.zeros((_L,), jnp.float32))

    pltpu.sync_copy(outv, out_hbm.at[pl.ds(base, B_PER_W)])


def kernel(user, business, user_factors, business_factors):
    return _mf_kernel(user, business, user_factors.T, business_factors.T)


# speedup vs baseline: 24.4660x; 1.1651x over previous
import functools

import jax
import jax.numpy as jnp
from jax import lax
from jax.experimental import pallas as pl
from jax.experimental.pallas import tpu as pltpu
from jax.experimental.pallas import tpu_sc as plsc

BATCH = 16384
NF = 32
TILE = 128
FETCH = 128
PITCH = 128

_info = plsc.get_sparse_core_info()
_NC, _NS, _L = _info.num_cores, _info.num_subcores, _info.num_lanes
NW = _NC * _NS
B_PER_W = BATCH // NW
RING = 8

_mesh = plsc.VectorSubcoreMesh(core_axis_name="c", subcore_axis_name="s")

_scratch = [
    pltpu.VMEM((B_PER_W,), jnp.int32),
    pltpu.VMEM((B_PER_W,), jnp.int32),
    pltpu.SMEM((B_PER_W,), jnp.int32),
    pltpu.SMEM((B_PER_W,), jnp.int32),
    pltpu.VMEM((B_PER_W,), jnp.float32),
]
for _ in range(RING):
    _scratch.append(pltpu.VMEM((NF, PITCH), jnp.float32))
    _scratch.append(pltpu.VMEM((NF, PITCH), jnp.float32))
    _scratch.append(pltpu.SemaphoreType.DMA)


@functools.partial(
    pl.kernel,
    mesh=_mesh,
    out_type=jax.ShapeDtypeStruct((BATCH,), jnp.float32),
    compiler_params=pltpu.CompilerParams(needs_layout_passes=False),
    scratch_types=_scratch,
)
def _mf_kernel(user_hbm, business_hbm, uft_hbm, bft_hbm, out_hbm,
               uidx_v, bidx_v, uidx, bidx, outv, *ring):
    wid = lax.axis_index("s") * _NC + lax.axis_index("c")
    base = wid * B_PER_W

    pltpu.sync_copy(user_hbm.at[pl.ds(base, B_PER_W)], uidx_v)
    pltpu.sync_copy(business_hbm.at[pl.ds(base, B_PER_W)], bidx_v)

    for g in range(B_PER_W // _L):
        vu = uidx_v[pl.ds(g * _L, _L)]
        vb = bidx_v[pl.ds(g * _L, _L)]
        for m in range(_L):
            uidx[g * _L + m] = vu[m]
            bidx[g * _L + m] = vb[m]

    ubufs = ring[0::3]
    bbufs = ring[1::3]
    sems = ring[2::3]
    lanes = lax.iota(jnp.int32, _L)

    def start(i, b):
        ru = uidx[i]
        rb = bidx[i]
        tu = pl.multiple_of(lax.bitwise_and(ru, -FETCH), TILE)
        tb = pl.multiple_of(lax.bitwise_and(rb, -FETCH), TILE)
        pltpu.async_copy(uft_hbm.at[pl.ds(0, NF), pl.ds(tu, FETCH)],
                         ubufs[b].at[pl.ds(0, NF), pl.ds(0, FETCH)], sems[b])
        pltpu.async_copy(bft_hbm.at[pl.ds(0, NF), pl.ds(tb, FETCH)],
                         bbufs[b].at[pl.ds(0, NF), pl.ds(0, FETCH)], sems[b])

    def drain(b):
        dummy = uft_hbm.at[pl.ds(0, NF), pl.ds(0, FETCH)]
        pltpu.make_async_copy(
            dummy, ubufs[b].at[pl.ds(0, NF), pl.ds(0, FETCH)], sems[b]).wait()
        pltpu.make_async_copy(
            dummy, bbufs[b].at[pl.ds(0, NF), pl.ds(0, FETCH)], sems[b]).wait()

    def compute(i, b, acc):
        lu = lax.bitwise_and(uidx[i], FETCH - 1)
        lb = lax.bitwise_and(bidx[i], FETCH - 1)
        lu_v = jnp.full((_L,), lu, jnp.int32)
        lb_v = jnp.full((_L,), lb, jnp.int32)
        u_lo = plsc.load_gather(ubufs[b], [lanes, lu_v])
        u_hi = plsc.load_gather(ubufs[b], [lanes + _L, lu_v])
        b_lo = plsc.load_gather(bbufs[b], [lanes, lb_v])
        b_hi = plsc.load_gather(bbufs[b], [lanes + _L, lb_v])
        prod = u_lo * b_lo + u_hi * b_hi
        s = jnp.full((_L,), jnp.sum(prod), jnp.float32)
        return jnp.where(lanes == lax.bitwise_and(i, _L - 1), s, acc)

    for b in range(RING):
        start(b, b)

    def macro_body(k, acc):
        for b in range(RING):
            i = k * RING + b
            drain(b)
            acc = compute(i, b, acc)

            @pl.when(i + RING < B_PER_W)
            def _():
                start(i + RING, b)

            flush = lax.bitwise_and(i, _L - 1) == _L - 1

            @pl.when(flush)
            def _():
                outv[pl.ds(lax.bitwise_and(i, -_L), _L)] = acc

            acc = jnp.where(flush, jnp.zeros((_L,), jnp.float32), acc)
        return acc

    lax.fori_loop(0, B_PER_W // RING, macro_body,
                  jnp.zeros((_L,), jnp.float32))

    pltpu.sync_copy(outv, out_hbm.at[pl.ds(base, B_PER_W)])


def kernel(user, business, user_factors, business_factors):
    return _mf_kernel(user, business, user_factors.T, business_factors.T)


# speedup vs baseline: 24.5657x; 1.0041x over previous
import functools

import jax
import jax.numpy as jnp
from jax import lax
from jax.experimental import pallas as pl
from jax.experimental.pallas import tpu as pltpu
from jax.experimental.pallas import tpu_sc as plsc

BATCH = 16384
NF = 32
TILE = 128
FETCH = 128
PITCH = 128

_info = plsc.get_sparse_core_info()
_NC, _NS, _L = _info.num_cores, _info.num_subcores, _info.num_lanes
NW = _NC * _NS
B_PER_W = BATCH // NW
RING = 8

_mesh = plsc.VectorSubcoreMesh(core_axis_name="c", subcore_axis_name="s")

_scratch = [
    pltpu.VMEM((B_PER_W,), jnp.int32),
    pltpu.VMEM((B_PER_W,), jnp.int32),
    pltpu.SMEM((B_PER_W,), jnp.int32),
    pltpu.SMEM((B_PER_W,), jnp.int32),
    pltpu.VMEM((B_PER_W,), jnp.float32),
]
for _ in range(RING):
    _scratch.append(pltpu.VMEM((NF, PITCH), jnp.float32))
    _scratch.append(pltpu.VMEM((NF, PITCH), jnp.float32))
    _scratch.append(pltpu.SemaphoreType.DMA)


@functools.partial(
    pl.kernel,
    mesh=_mesh,
    out_type=jax.ShapeDtypeStruct((BATCH,), jnp.float32),
    compiler_params=pltpu.CompilerParams(needs_layout_passes=False),
    scratch_types=_scratch,
)
def _mf_kernel(user_hbm, business_hbm, uft_hbm, bft_hbm, out_hbm,
               uidx_v, bidx_v, uidx, bidx, outv, *ring):
    wid = lax.axis_index("s") * _NC + lax.axis_index("c")
    base = wid * B_PER_W

    pltpu.sync_copy(user_hbm.at[pl.ds(base, B_PER_W)], uidx_v)
    pltpu.sync_copy(business_hbm.at[pl.ds(base, B_PER_W)], bidx_v)

    for g in range(B_PER_W // _L):
        vu = uidx_v[pl.ds(g * _L, _L)]
        vb = bidx_v[pl.ds(g * _L, _L)]
        for m in range(_L):
            uidx[g * _L + m] = vu[m]
            bidx[g * _L + m] = vb[m]

    ubufs = ring[0::3]
    bbufs = ring[1::3]
    sems = ring[2::3]
    lanes = lax.iota(jnp.int32, _L)

    def start(i, b):
        ru = uidx[i]
        rb = bidx[i]
        tu = pl.multiple_of(lax.bitwise_and(ru, -FETCH), TILE)
        tb = pl.multiple_of(lax.bitwise_and(rb, -FETCH), TILE)
        for a in range(NF // 8):
            rows = pl.ds(8 * a, 8)
            pltpu.async_copy(uft_hbm.at[rows, pl.ds(tu, FETCH)],
                             ubufs[b].at[rows, pl.ds(0, FETCH)], sems[b])
            pltpu.async_copy(bft_hbm.at[rows, pl.ds(tb, FETCH)],
                             bbufs[b].at[rows, pl.ds(0, FETCH)], sems[b])

    def drain(b):
        dummy = uft_hbm.at[pl.ds(0, NF), pl.ds(0, FETCH)]
        pltpu.make_async_copy(
            dummy, ubufs[b].at[pl.ds(0, NF), pl.ds(0, FETCH)], sems[b]).wait()
        pltpu.make_async_copy(
            dummy, bbufs[b].at[pl.ds(0, NF), pl.ds(0, FETCH)], sems[b]).wait()

    def compute(i, b, acc):
        lu = lax.bitwise_and(uidx[i], FETCH - 1)
        lb = lax.bitwise_and(bidx[i], FETCH - 1)
        lu_v = jnp.full((_L,), lu, jnp.int32)
        lb_v = jnp.full((_L,), lb, jnp.int32)
        u_lo = plsc.load_gather(ubufs[b], [lanes, lu_v])
        u_hi = plsc.load_gather(ubufs[b], [lanes + _L, lu_v])
        b_lo = plsc.load_gather(bbufs[b], [lanes, lb_v])
        b_hi = plsc.load_gather(bbufs[b], [lanes + _L, lb_v])
        prod = u_lo * b_lo + u_hi * b_hi
        s = jnp.full((_L,), jnp.sum(prod), jnp.float32)
        return jnp.where(lanes == lax.bitwise_and(i, _L - 1), s, acc)

    for b in range(RING):
        start(b, b)

    def macro_body(k, acc):
        for b in range(RING):
            i = k * RING + b
            drain(b)
            acc = compute(i, b, acc)

            @pl.when(i + RING < B_PER_W)
            def _():
                start(i + RING, b)

            flush = lax.bitwise_and(i, _L - 1) == _L - 1

            @pl.when(flush)
            def _():
                outv[pl.ds(lax.bitwise_and(i, -_L), _L)] = acc

            acc = jnp.where(flush, jnp.zeros((_L,), jnp.float32), acc)
        return acc

    lax.fori_loop(0, B_PER_W // RING, macro_body,
                  jnp.zeros((_L,), jnp.float32))

    pltpu.sync_copy(outv, out_hbm.at[pl.ds(base, B_PER_W)])


def kernel(user, business, user_factors, business_factors):
    return _mf_kernel(user, business, user_factors.T, business_factors.T)
